# baseline (speedup 1.0000x reference)
import functools
import math

import jax
import jax.numpy as jnp
from jax import lax
from jax.experimental import pallas as pl
from jax.experimental.pallas import tpu as pltpu
from jax.experimental.pallas import tpu_sc as plsc

_LS = 0.1
_CONF = 1.0 - _LS
_BLK = 4096
_CS = 4096
_CH = 2048

_NC = 2
_NS = 16
_L = 16
_NW = _NC * _NS


def _sc_body(b, n, x2d, tgt, acc_out, tacc_out, tgt_v, buf, accbuf, tacc_v,
             sem0, sem1):
    s = _LS / (n - 2)
    sce = (n // 128) * 128
    w = sce - _CS
    nfull = (w // _CH) & ~1
    tail = []
    off = _CS + nfull * _CH
    while off < sce:
        wd = min(_CH, sce - off)
        tail.append((off, wd))
        off += wd
    rows_per = b // _NW
    wid = lax.axis_index("s") * _NC + lax.axis_index("c")
    base = wid * rows_per

    pltpu.sync_copy(tgt.at[pl.ds(base, rows_per)], tgt_v)
    tacc_v[...] = jnp.zeros((_L,), jnp.float32)

    def start(r0, k, bsel, sem):
        pltpu.make_async_copy(
            x2d.at[pl.ds(r0, 8), pl.ds(_CS + k * _CH, _CH)],
            buf.at[bsel], sem).start()

    def wait(bsel, sem):
        pltpu.make_async_copy(
            x2d.at[pl.ds(0, 8), pl.ds(0, _CH)], buf.at[bsel], sem).wait()

    def compute(bufref, c0, cw, rg):
        for r in range(8):
            nv = cw // _L

            @pl.loop(0, nv // 8, init_carry=(
                jnp.zeros((_L,), jnp.float32), jnp.zeros((_L,), jnp.float32),
                jnp.zeros((_L,), jnp.float32), jnp.zeros((_L,), jnp.float32)))
            def chains(i, carry):
                a0, a1, a2, a3 = carry
                o = i * (8 * _L)
                a0 = a0 + bufref[r, pl.ds(o, _L)]
                a1 = a1 + bufref[r, pl.ds(o + _L, _L)]
                a2 = a2 + bufref[r, pl.ds(o + 2 * _L, _L)]
                a3 = a3 + bufref[r, pl.ds(o + 3 * _L, _L)]
                a0 = a0 + bufref[r, pl.ds(o + 4 * _L, _L)]
                a1 = a1 + bufref[r, pl.ds(o + 5 * _L, _L)]
                a2 = a2 + bufref[r, pl.ds(o + 6 * _L, _L)]
                a3 = a3 + bufref[r, pl.ds(o + 7 * _L, _L)]
                return a0, a1, a2, a3

            a0, a1, a2, a3 = chains
            accbuf[r] = accbuf[r] + (a0 + a1) + (a2 + a3)

            lr = jnp.full((_L,), rg * 8 + r, jnp.int32)
            t_spl = plsc.load_gather(tgt_v, [lr])
            idx = t_spl - c0
            lane0 = lax.iota(jnp.int32, _L) == 0
            valid = ((t_spl >= c0) & (t_spl < c0 + cw) & lane0
                     & (t_spl != 0))
            idx_c = jnp.minimum(jnp.maximum(idx, 0), cw - 1)
            xt = plsc.load_gather(bufref, [jnp.full((_L,), r, jnp.int32),
                                           idx_c], mask=valid)
            zero = jnp.zeros((_L,), jnp.float32)
            wvec = jnp.full((_L,), s - _CONF, jnp.float32)
            tacc_v[...] = tacc_v[...] + jnp.where(valid, wvec * xt, zero)

    for rg in range(4):
        r0 = base + rg * 8
        for r in range(8):
            accbuf[r] = jnp.zeros((_L,), jnp.float32)
        if nfull > 0:
            start(r0, 0, 0, sem0)
        if nfull > 1:
            start(r0, 1, 1, sem1)

        @pl.loop(0, nfull, step=2)
        def pair(k):
            wait(0, sem0)
            compute(buf.at[0], _CS + k * _CH, _CH, rg)

            @pl.when(k + 2 < nfull)
            def _():
                start(r0, k + 2, 0, sem0)

            wait(1, sem1)
            compute(buf.at[1], _CS + (k + 1) * _CH, _CH, rg)

            @pl.when(k + 3 < nfull)
            def _():
                start(r0, k + 3, 1, sem1)

        for (toff, twd) in tail:
            pltpu.sync_copy(x2d.at[pl.ds(r0, 8), pl.ds(toff, twd)],
                            buf.at[0, :, pl.ds(0, twd)])
            compute(buf.at[0], toff, twd, rg)

        pltpu.sync_copy(accbuf, acc_out.at[pl.ds(r0, 8)])

    pltpu.sync_copy(tacc_v, tacc_out.at[wid])


def _sc_stream(output, tgt):
    b, n = output.shape
    mesh = plsc.VectorSubcoreMesh(core_axis_name="c", subcore_axis_name="s",
                                  num_cores=_NC, num_subcores=_NS)
    body = functools.partial(_sc_body, b, n)
    return pl.kernel(
        body,
        out_type=(jax.ShapeDtypeStruct((b, _L), jnp.float32),
                  jax.ShapeDtypeStruct((_NW, _L), jnp.float32)),
        mesh=mesh,
        compiler_params=pltpu.CompilerParams(needs_layout_passes=False),
        scratch_types=[
            pltpu.VMEM((b // _NW,), jnp.int32),
            pltpu.VMEM((2, 8, _CH), jnp.float32),
            pltpu.VMEM((8, _L), jnp.float32),
            pltpu.VMEM((_L,), jnp.float32),
            pltpu.SemaphoreType.DMA,
            pltpu.SemaphoreType.DMA,
        ],
    )(output, tgt)


def _tc_body(nblocks, n, t_ref, x_ref, acc_ref, tacc_ref, zacc_ref):
    j = pl.program_id(0)

    @pl.when(j == 0)
    def _init():
        acc_ref[...] = jnp.zeros_like(acc_ref)
        tacc_ref[...] = jnp.zeros_like(tacc_ref)

    x = x_ref[...]
    acc_ref[...] += jnp.sum(x, axis=1, keepdims=True)
    cols = j * _BLK + jax.lax.broadcasted_iota(jnp.int32, x.shape, 1)
    t = t_ref[...]
    tacc_ref[...] += jnp.sum(jnp.where(cols == t, x, jnp.zeros_like(x)),
                             axis=1, keepdims=True)

    @pl.when(j == 0)
    def _zcol():
        zacc_ref[...] = x[:, 0:1]


def _tc_stream(t2, output):
    b, n = output.shape
    nblocks = _CS // _BLK
    body = functools.partial(_tc_body, nblocks, n)
    return pl.pallas_call(
        body,
        grid=(nblocks,),
        in_specs=[
            pl.BlockSpec((b, 1), lambda j: (0, 0)),
            pl.BlockSpec((b, _BLK), lambda j: (0, j)),
        ],
        out_specs=[
            pl.BlockSpec((b, 1), lambda j: (0, 0)),
            pl.BlockSpec((b, 1), lambda j: (0, 0)),
            pl.BlockSpec((b, 1), lambda j: (0, 0)),
        ],
        out_shape=[
            jax.ShapeDtypeStruct((b, 1), jnp.float32),
            jax.ShapeDtypeStruct((b, 1), jnp.float32),
            jax.ShapeDtypeStruct((b, 1), jnp.float32),
        ],
    )(t2, output)


def _combine_body(n, t_ref, tail_ref, acc_tc, tacc_tc, zacc, acc_sc, tacc_sc,
                  out_ref):
    s = _LS / (n - 2)
    k_const = _LS * math.log(s) + _CONF * math.log(_CONF)
    base = (n // 128) * 128
    xtail = tail_ref[...]
    cols = base + jax.lax.broadcasted_iota(jnp.int32, xtail.shape, 1)
    t = t_ref[...]
    zero = jnp.zeros_like(xtail)
    rowsum_tail = jnp.sum(jnp.where(cols < n, xtail, zero), axis=1,
                          keepdims=True)
    xt_tail = jnp.sum(jnp.where(cols == t, xtail, zero), axis=1,
                      keepdims=True)
    r_total = acc_tc[...] + rowsum_tail + jnp.sum(acc_sc[...], axis=1,
                                                  keepdims=True)
    xt = tacc_tc[...] + xt_tail
    contrib = k_const - s * r_total + s * zacc[...] + (s - _CONF) * xt
    nonpad = t != 0
    total = jnp.sum(jnp.where(nonpad, contrib, jnp.zeros_like(contrib)))
    total += jnp.sum(tacc_sc[...])
    out_ref[...] = total.reshape(1, 1)


def _combine(t2, output, acc_tc, tacc_tc, zacc, acc_sc, tacc_sc):
    b, n = output.shape
    body = functools.partial(_combine_body, n)
    return pl.pallas_call(
        body,
        grid=(1,),
        in_specs=[
            pl.BlockSpec((b, 1), lambda j: (0, 0)),
            pl.BlockSpec((b, 128), lambda j: (0, n // 128)),
            pl.BlockSpec((b, 1), lambda j: (0, 0)),
            pl.BlockSpec((b, 1), lambda j: (0, 0)),
            pl.BlockSpec((b, 1), lambda j: (0, 0)),
            pl.BlockSpec((b, _L), lambda j: (0, 0)),
            pl.BlockSpec((_NW, _L), lambda j: (0, 0)),
        ],
        out_specs=pl.BlockSpec((1, 1), lambda j: (0, 0)),
        out_shape=jax.ShapeDtypeStruct((1, 1), jnp.float32),
    )(t2, output, acc_tc, tacc_tc, zacc, acc_sc, tacc_sc)


def kernel(output, target, extra_len):
    del extra_len
    b, n = output.shape
    tgt = target.astype(jnp.int32)
    t2 = tgt.reshape(b, 1)
    acc_sc, tacc_sc = _sc_stream(output, tgt)
    acc_tc, tacc_tc, zacc = _tc_stream(t2, output)
    res = _combine(t2, output, acc_tc, tacc_tc, zacc, acc_sc, tacc_sc)
    return res[0, 0]


# speedup vs baseline: 1.1139x; 1.1139x over previous
import functools
import math

import jax
import jax.numpy as jnp
from jax import lax
from jax.experimental import pallas as pl
from jax.experimental.pallas import tpu as pltpu
from jax.experimental.pallas import tpu_sc as plsc

_LS = 0.1
_CONF = 1.0 - _LS
_BLK = 4096
_CS = 4096
_CE = 77824
_CH = 4096

_NC = 2
_NS = 16
_L = 16
_NW = _NC * _NS


def _sc_body(b, n, x2d, tgt, acc_out, tacc_out, tgt_v, buf, accbuf, tacc_v,
             sem0, sem1):
    s = _LS / (n - 2)
    w = _CE - _CS
    nfull = (w // _CH) & ~1
    tail = []
    off = _CS + nfull * _CH
    while off < _CE:
        wd = min(_CH, _CE - off)
        tail.append((off, wd))
        off += wd
    rows_per = b // _NW
    wid = lax.axis_index("s") * _NC + lax.axis_index("c")
    base = wid * rows_per

    pltpu.sync_copy(tgt.at[pl.ds(base, rows_per)], tgt_v)
    tacc_v[...] = jnp.zeros((_L,), jnp.float32)

    def start(r0, k, bsel, sem):
        pltpu.make_async_copy(
            x2d.at[pl.ds(r0, 8), pl.ds(_CS + k * _CH, _CH)],
            buf.at[bsel], sem).start()

    def wait(bsel, sem):
        pltpu.make_async_copy(
            x2d.at[pl.ds(0, 8), pl.ds(0, _CH)], buf.at[bsel], sem).wait()

    def compute(bufref, c0, cw, rg):
        for r in range(8):
            nv = cw // _L

            @pl.loop(0, nv // 8, init_carry=(
                jnp.zeros((_L,), jnp.float32), jnp.zeros((_L,), jnp.float32),
                jnp.zeros((_L,), jnp.float32), jnp.zeros((_L,), jnp.float32)))
            def chains(i, carry):
                a0, a1, a2, a3 = carry
                o = i * (8 * _L)
                a0 = a0 + bufref[r, pl.ds(o, _L)]
                a1 = a1 + bufref[r, pl.ds(o + _L, _L)]
                a2 = a2 + bufref[r, pl.ds(o + 2 * _L, _L)]
                a3 = a3 + bufref[r, pl.ds(o + 3 * _L, _L)]
                a0 = a0 + bufref[r, pl.ds(o + 4 * _L, _L)]
                a1 = a1 + bufref[r, pl.ds(o + 5 * _L, _L)]
                a2 = a2 + bufref[r, pl.ds(o + 6 * _L, _L)]
                a3 = a3 + bufref[r, pl.ds(o + 7 * _L, _L)]
                return a0, a1, a2, a3

            a0, a1, a2, a3 = chains
            accbuf[r] = accbuf[r] + (a0 + a1) + (a2 + a3)

            lr = jnp.full((_L,), rg * 8 + r, jnp.int32)
            t_spl = plsc.load_gather(tgt_v, [lr])
            idx = t_spl - c0
            lane0 = lax.iota(jnp.int32, _L) == 0
            valid = ((t_spl >= c0) & (t_spl < c0 + cw) & lane0
                     & (t_spl != 0))
            idx_c = jnp.minimum(jnp.maximum(idx, 0), cw - 1)
            xt = plsc.load_gather(bufref, [jnp.full((_L,), r, jnp.int32),
                                           idx_c], mask=valid)
            zero = jnp.zeros((_L,), jnp.float32)
            wvec = jnp.full((_L,), s - _CONF, jnp.float32)
            tacc_v[...] = tacc_v[...] + jnp.where(valid, wvec * xt, zero)

    for rg in range(4):
        r0 = base + rg * 8
        for r in range(8):
            accbuf[r] = jnp.zeros((_L,), jnp.float32)
        if nfull > 0:
            start(r0, 0, 0, sem0)
        if nfull > 1:
            start(r0, 1, 1, sem1)

        @pl.loop(0, nfull, step=2)
        def pair(k):
            wait(0, sem0)
            compute(buf.at[0], _CS + k * _CH, _CH, rg)

            @pl.when(k + 2 < nfull)
            def _():
                start(r0, k + 2, 0, sem0)

            wait(1, sem1)
            compute(buf.at[1], _CS + (k + 1) * _CH, _CH, rg)

            @pl.when(k + 3 < nfull)
            def _():
                start(r0, k + 3, 1, sem1)

        for (toff, twd) in tail:
            pltpu.sync_copy(x2d.at[pl.ds(r0, 8), pl.ds(toff, twd)],
                            buf.at[0, :, pl.ds(0, twd)])
            compute(buf.at[0], toff, twd, rg)

        pltpu.sync_copy(accbuf, acc_out.at[pl.ds(r0, 8)])

    pltpu.sync_copy(tacc_v, tacc_out.at[wid])


def _sc_stream(output, tgt):
    b, n = output.shape
    mesh = plsc.VectorSubcoreMesh(core_axis_name="c", subcore_axis_name="s",
                                  num_cores=_NC, num_subcores=_NS)
    body = functools.partial(_sc_body, b, n)
    return pl.kernel(
        body,
        out_type=(jax.ShapeDtypeStruct((b, _L), jnp.float32),
                  jax.ShapeDtypeStruct((_NW, _L), jnp.float32)),
        mesh=mesh,
        compiler_params=pltpu.CompilerParams(needs_layout_passes=False),
        scratch_types=[
            pltpu.VMEM((b // _NW,), jnp.int32),
            pltpu.VMEM((2, 8, _CH), jnp.float32),
            pltpu.VMEM((8, _L), jnp.float32),
            pltpu.VMEM((_L,), jnp.float32),
            pltpu.SemaphoreType.DMA,
            pltpu.SemaphoreType.DMA,
        ],
    )(output, tgt)


def _tc_body(nblocks, n, t_ref, x_ref, acc_ref, tacc_ref, zacc_ref):
    j = pl.program_id(0)

    @pl.when(j == 0)
    def _init():
        acc_ref[...] = jnp.zeros_like(acc_ref)
        tacc_ref[...] = jnp.zeros_like(tacc_ref)

    x = x_ref[...]
    c0 = jnp.where(j == 0, 0, _CE + (j - 1) * _BLK)
    cols = c0 + jax.lax.broadcasted_iota(jnp.int32, x.shape, 1)
    t = t_ref[...]
    zero = jnp.zeros_like(x)
    xv = jnp.where(cols < n, x, zero)
    acc_ref[...] += jnp.sum(xv, axis=1, keepdims=True)
    tacc_ref[...] += jnp.sum(jnp.where(cols == t, x, zero), axis=1,
                             keepdims=True)

    @pl.when(j == 0)
    def _zcol():
        zacc_ref[...] = x[:, 0:1]


def _tc_stream(t2, output):
    b, n = output.shape
    nback = pl.cdiv(n - _CE, _BLK)
    nblocks = 1 + nback
    ce_blk = _CE // _BLK
    body = functools.partial(_tc_body, nblocks, n)
    return pl.pallas_call(
        body,
        grid=(nblocks,),
        in_specs=[
            pl.BlockSpec((b, 1), lambda j: (0, 0)),
            pl.BlockSpec((b, _BLK),
                         lambda j: (0, jnp.where(j == 0, 0, ce_blk + j - 1))),
        ],
        out_specs=[
            pl.BlockSpec((b, 1), lambda j: (0, 0)),
            pl.BlockSpec((b, 1), lambda j: (0, 0)),
            pl.BlockSpec((b, 1), lambda j: (0, 0)),
        ],
        out_shape=[
            jax.ShapeDtypeStruct((b, 1), jnp.float32),
            jax.ShapeDtypeStruct((b, 1), jnp.float32),
            jax.ShapeDtypeStruct((b, 1), jnp.float32),
        ],
    )(t2, output)


def _combine_body(n, t_ref, acc_tc, tacc_tc, zacc, acc_sc, tacc_sc, out_ref):
    s = _LS / (n - 2)
    k_const = _LS * math.log(s) + _CONF * math.log(_CONF)
    t = t_ref[...]
    r_total = acc_tc[...] + jnp.sum(acc_sc[...], axis=1, keepdims=True)
    xt = tacc_tc[...]
    contrib = k_const - s * r_total + s * zacc[...] + (s - _CONF) * xt
    nonpad = t != 0
    total = jnp.sum(jnp.where(nonpad, contrib, jnp.zeros_like(contrib)))
    total += jnp.sum(tacc_sc[...])
    out_ref[...] = total.reshape(1, 1)


def kernel(output, target, extra_len):
    del extra_len
    b, n = output.shape
    tgt = target.astype(jnp.int32)
    t2 = tgt.reshape(b, 1)
    acc_sc, tacc_sc = _sc_stream(output, tgt)
    acc_tc, tacc_tc, zacc = _tc_stream(t2, output)
    body = functools.partial(_combine_body, n)
    res = pl.pallas_call(
        body,
        grid=(1,),
        in_specs=[
            pl.BlockSpec((b, 1), lambda j: (0, 0)),
            pl.BlockSpec((b, 1), lambda j: (0, 0)),
            pl.BlockSpec((b, 1), lambda j: (0, 0)),
            pl.BlockSpec((b, 1), lambda j: (0, 0)),
            pl.BlockSpec((b, _L), lambda j: (0, 0)),
            pl.BlockSpec((_NW, _L), lambda j: (0, 0)),
        ],
        out_specs=pl.BlockSpec((1, 1), lambda j: (0, 0)),
        out_shape=jax.ShapeDtypeStruct((1, 1), jnp.float32),
    )(t2, acc_tc, tacc_tc, zacc, acc_sc, tacc_sc)
    return res[0, 0]


# speedup vs baseline: 3.7048x; 3.3259x over previous
import functools
import math

import jax
import jax.numpy as jnp
from jax import lax
from jax.experimental import pallas as pl
from jax.experimental.pallas import tpu as pltpu
from jax.experimental.pallas import tpu_sc as plsc

_LS = 0.1
_CONF = 1.0 - _LS
_RBLK = 2048
_A = 2048
_B = 51200
_CHR = 32

_NC = 2
_NS = 16
_L = 16
_NW = _NC * _NS


def _sc_body(b, n, xT, tgt, acc_out, tacc_out, tgt_v, buf, acc_v, tacc_v,
             sem0, sem1):
    w_per = (_B - _A) // _NW
    nchunks = w_per // _CHR
    ngroups = b // _L
    wid = lax.axis_index("s") * _NC + lax.axis_index("c")
    c_base = _A + wid * w_per

    pltpu.sync_copy(tgt, tgt_v)

    @pl.loop(0, ngroups)
    def _zero(j):
        off = j * _L
        acc_v[pl.ds(off, _L)] = jnp.zeros((_L,), jnp.float32)
        tacc_v[pl.ds(off, _L)] = jnp.zeros((_L,), jnp.float32)

    def start(k, bsel, sem):
        pltpu.make_async_copy(
            xT.at[pl.ds(c_base + k * _CHR, _CHR), :],
            buf.at[bsel], sem).start()

    def wait(bsel, sem):
        pltpu.make_async_copy(
            xT.at[pl.ds(0, _CHR), :], buf.at[bsel], sem).wait()

    def compute(bufref, c0):
        @pl.loop(0, ngroups)
        def _grp(j):
            off = j * _L
            a0 = bufref[0, pl.ds(off, _L)]
            a1 = bufref[1, pl.ds(off, _L)]
            a2 = bufref[2, pl.ds(off, _L)]
            a3 = bufref[3, pl.ds(off, _L)]
            for r in range(4, _CHR, 4):
                a0 = a0 + bufref[r, pl.ds(off, _L)]
                a1 = a1 + bufref[r + 1, pl.ds(off, _L)]
                a2 = a2 + bufref[r + 2, pl.ds(off, _L)]
                a3 = a3 + bufref[r + 3, pl.ds(off, _L)]
            acc_v[pl.ds(off, _L)] = (acc_v[pl.ds(off, _L)]
                                     + (a0 + a1) + (a2 + a3))

            t16 = tgt_v[pl.ds(off, _L)]
            rel = t16 - c0
            valid = (rel >= 0) & (rel < _CHR)
            relc = jnp.minimum(jnp.maximum(rel, 0), _CHR - 1)
            bcol = off + lax.iota(jnp.int32, _L)
            xt = plsc.load_gather(bufref, [relc, bcol])
            zero = jnp.zeros((_L,), jnp.float32)
            tacc_v[pl.ds(off, _L)] = (tacc_v[pl.ds(off, _L)]
                                      + jnp.where(valid, xt, zero))

    if nchunks > 0:
        start(0, 0, sem0)
    if nchunks > 1:
        start(1, 1, sem1)

    @pl.loop(0, nchunks, step=2)
    def pair(k):
        wait(0, sem0)
        compute(buf.at[0], c_base + k * _CHR)

        @pl.when(k + 2 < nchunks)
        def _():
            start(k + 2, 0, sem0)

        wait(1, sem1)
        compute(buf.at[1], c_base + (k + 1) * _CHR)

        @pl.when(k + 3 < nchunks)
        def _():
            start(k + 3, 1, sem1)

    pltpu.sync_copy(acc_v, acc_out.at[wid])
    pltpu.sync_copy(tacc_v, tacc_out.at[wid])


def _sc_stream(xT, tgt):
    n, b = xT.shape
    mesh = plsc.VectorSubcoreMesh(core_axis_name="c", subcore_axis_name="s",
                                  num_cores=_NC, num_subcores=_NS)
    body = functools.partial(_sc_body, b, n)
    return pl.kernel(
        body,
        out_type=(jax.ShapeDtypeStruct((_NW, b), jnp.float32),
                  jax.ShapeDtypeStruct((_NW, b), jnp.float32)),
        mesh=mesh,
        compiler_params=pltpu.CompilerParams(needs_layout_passes=False),
        scratch_types=[
            pltpu.VMEM((b,), jnp.int32),
            pltpu.VMEM((2, _CHR, b), jnp.float32),
            pltpu.VMEM((b,), jnp.float32),
            pltpu.VMEM((b,), jnp.float32),
            pltpu.SemaphoreType.DMA,
            pltpu.SemaphoreType.DMA,
        ],
    )(xT, tgt)


def _tc_body(n, t_ref, x_ref, acc_ref, tacc_ref, zacc_ref):
    j = pl.program_id(0)

    @pl.when(j == 0)
    def _init():
        acc_ref[...] = jnp.zeros_like(acc_ref)
        tacc_ref[...] = jnp.zeros_like(tacc_ref)

    x = x_ref[...]
    c0 = jnp.where(j == 0, 0, _B + (j - 1) * _RBLK)
    rid = c0 + jax.lax.broadcasted_iota(jnp.int32, x.shape, 0)
    t = t_ref[...]
    zero = jnp.zeros_like(x)
    xv = jnp.where(rid < n, x, zero)
    acc_ref[...] += jnp.sum(xv, axis=0, keepdims=True)
    tacc_ref[...] += jnp.sum(jnp.where(rid == t, x, zero), axis=0,
                             keepdims=True)

    @pl.when(j == 0)
    def _zrow():
        zacc_ref[...] = x[0:1, :]


def _tc_stream(t_row, xT):
    n, b = xT.shape
    nback = pl.cdiv(n - _B, _RBLK)
    nblocks = 1 + nback
    b_blk = _B // _RBLK
    body = functools.partial(_tc_body, n)
    return pl.pallas_call(
        body,
        grid=(nblocks,),
        in_specs=[
            pl.BlockSpec((1, b), lambda j: (0, 0)),
            pl.BlockSpec((_RBLK, b),
                         lambda j: (jnp.where(j == 0, 0, b_blk + j - 1), 0)),
        ],
        out_specs=[
            pl.BlockSpec((1, b), lambda j: (0, 0)),
            pl.BlockSpec((1, b), lambda j: (0, 0)),
            pl.BlockSpec((1, b), lambda j: (0, 0)),
        ],
        out_shape=[
            jax.ShapeDtypeStruct((1, b), jnp.float32),
            jax.ShapeDtypeStruct((1, b), jnp.float32),
            jax.ShapeDtypeStruct((1, b), jnp.float32),
        ],
    )(t_row, xT)


def _combine_body(n, t_ref, acc_tc, tacc_tc, zacc, acc_sc, tacc_sc, out_ref):
    s = _LS / (n - 2)
    k_const = _LS * math.log(s) + _CONF * math.log(_CONF)
    t = t_ref[...]
    r_total = acc_tc[...] + jnp.sum(acc_sc[...], axis=0, keepdims=True)
    xt = tacc_tc[...] + jnp.sum(tacc_sc[...], axis=0, keepdims=True)
    contrib = k_const - s * r_total + s * zacc[...] + (s - _CONF) * xt
    nonpad = t != 0
    total = jnp.sum(jnp.where(nonpad, contrib, jnp.zeros_like(contrib)))
    out_ref[...] = total.reshape(1, 1)


def kernel(output, target, extra_len):
    del extra_len
    b, n = output.shape
    xT = output.T
    tgt = target.astype(jnp.int32)
    t_row = tgt.reshape(1, b)
    acc_sc, tacc_sc = _sc_stream(xT, tgt)
    acc_tc, tacc_tc, zacc = _tc_stream(t_row, xT)
    body = functools.partial(_combine_body, n)
    res = pl.pallas_call(
        body,
        grid=(1,),
        in_specs=[
            pl.BlockSpec((1, b), lambda j: (0, 0)),
            pl.BlockSpec((1, b), lambda j: (0, 0)),
            pl.BlockSpec((1, b), lambda j: (0, 0)),
            pl.BlockSpec((1, b), lambda j: (0, 0)),
            pl.BlockSpec((_NW, b), lambda j: (0, 0)),
            pl.BlockSpec((_NW, b), lambda j: (0, 0)),
        ],
        out_specs=pl.BlockSpec((1, 1), lambda j: (0, 0)),
        out_shape=jax.ShapeDtypeStruct((1, 1), jnp.float32),
    )(t_row, acc_tc, tacc_tc, zacc, acc_sc, tacc_sc)
    return res[0, 0]


# speedup vs baseline: 3.9454x; 1.0649x over previous
import functools
import math

import jax
import jax.numpy as jnp
from jax import lax
from jax.experimental import pallas as pl
from jax.experimental.pallas import tpu as pltpu
from jax.experimental.pallas import tpu_sc as plsc

_LS = 0.1
_CONF = 1.0 - _LS
_RBLK = 2048
_A = 2048
_B = 47104
_CHR = 32

_NC = 2
_NS = 16
_L = 16
_NW = _NC * _NS


def _sc_body(b, n, xT, tgt, acc_out, tacc_out, tgt_v, buf, acc_v, tacc_v,
             sem0, sem1):
    w_per = (_B - _A) // _NW
    nchunks = w_per // _CHR
    ngroups = b // _L
    wid = lax.axis_index("s") * _NC + lax.axis_index("c")
    c_base = _A + wid * w_per

    pltpu.sync_copy(tgt, tgt_v)

    @pl.loop(0, ngroups)
    def _zero(j):
        off = j * _L
        acc_v[pl.ds(off, _L)] = jnp.zeros((_L,), jnp.float32)
        tacc_v[pl.ds(off, _L)] = jnp.zeros((_L,), jnp.float32)

    def start(k, bsel, sem):
        pltpu.make_async_copy(
            xT.at[pl.ds(c_base + k * _CHR, _CHR), :],
            buf.at[bsel], sem).start()

    def wait(bsel, sem):
        pltpu.make_async_copy(
            xT.at[pl.ds(0, _CHR), :], buf.at[bsel], sem).wait()

    def compute(bufref, c0):
        @pl.loop(0, ngroups)
        def _grp(j):
            off = j * _L
            a0 = bufref[0, pl.ds(off, _L)]
            a1 = bufref[1, pl.ds(off, _L)]
            a2 = bufref[2, pl.ds(off, _L)]
            a3 = bufref[3, pl.ds(off, _L)]
            for r in range(4, _CHR, 4):
                a0 = a0 + bufref[r, pl.ds(off, _L)]
                a1 = a1 + bufref[r + 1, pl.ds(off, _L)]
                a2 = a2 + bufref[r + 2, pl.ds(off, _L)]
                a3 = a3 + bufref[r + 3, pl.ds(off, _L)]
            acc_v[pl.ds(off, _L)] = (acc_v[pl.ds(off, _L)]
                                     + (a0 + a1) + (a2 + a3))

            t16 = tgt_v[pl.ds(off, _L)]
            rel = t16 - c0
            valid = (rel >= 0) & (rel < _CHR)
            relc = jnp.minimum(jnp.maximum(rel, 0), _CHR - 1)
            bcol = off + lax.iota(jnp.int32, _L)
            xt = plsc.load_gather(bufref, [relc, bcol])
            zero = jnp.zeros((_L,), jnp.float32)
            tacc_v[pl.ds(off, _L)] = (tacc_v[pl.ds(off, _L)]
                                      + jnp.where(valid, xt, zero))

    if nchunks > 0:
        start(0, 0, sem0)
    if nchunks > 1:
        start(1, 1, sem1)

    @pl.loop(0, nchunks, step=2)
    def pair(k):
        wait(0, sem0)
        compute(buf.at[0], c_base + k * _CHR)

        @pl.when(k + 2 < nchunks)
        def _():
            start(k + 2, 0, sem0)

        wait(1, sem1)
        compute(buf.at[1], c_base + (k + 1) * _CHR)

        @pl.when(k + 3 < nchunks)
        def _():
            start(k + 3, 1, sem1)

    pltpu.sync_copy(acc_v, acc_out.at[wid])
    pltpu.sync_copy(tacc_v, tacc_out.at[wid])


def _sc_stream(xT, tgt):
    n, b = xT.shape
    mesh = plsc.VectorSubcoreMesh(core_axis_name="c", subcore_axis_name="s",
                                  num_cores=_NC, num_subcores=_NS)
    body = functools.partial(_sc_body, b, n)
    return pl.kernel(
        body,
        out_type=(jax.ShapeDtypeStruct((_NW, b), jnp.float32),
                  jax.ShapeDtypeStruct((_NW, b), jnp.float32)),
        mesh=mesh,
        compiler_params=pltpu.CompilerParams(needs_layout_passes=False),
        scratch_types=[
            pltpu.VMEM((b,), jnp.int32),
            pltpu.VMEM((2, _CHR, b), jnp.float32),
            pltpu.VMEM((b,), jnp.float32),
            pltpu.VMEM((b,), jnp.float32),
            pltpu.SemaphoreType.DMA,
            pltpu.SemaphoreType.DMA,
        ],
    )(xT, tgt)


def _tc_body(n, t_ref, x_ref, acc_ref, tacc_ref, zacc_ref):
    j = pl.program_id(0)

    @pl.when(j == 0)
    def _init():
        acc_ref[...] = jnp.zeros_like(acc_ref)
        tacc_ref[...] = jnp.zeros_like(tacc_ref)

    x = x_ref[...]
    c0 = jnp.where(j == 0, 0, _B + (j - 1) * _RBLK)
    rid = c0 + jax.lax.broadcasted_iota(jnp.int32, x.shape, 0)
    t = t_ref[...]
    zero = jnp.zeros_like(x)
    xv = jnp.where(rid < n, x, zero)
    acc_ref[...] += jnp.sum(xv, axis=0, keepdims=True)
    tacc_ref[...] += jnp.sum(jnp.where(rid == t, x, zero), axis=0,
                             keepdims=True)

    @pl.when(j == 0)
    def _zrow():
        zacc_ref[...] = x[0:1, :]


def _tc_stream(t_row, xT):
    n, b = xT.shape
    nback = pl.cdiv(n - _B, _RBLK)
    nblocks = 1 + nback
    b_blk = _B // _RBLK
    body = functools.partial(_tc_body, n)
    return pl.pallas_call(
        body,
        grid=(nblocks,),
        in_specs=[
            pl.BlockSpec((1, b), lambda j: (0, 0)),
            pl.BlockSpec((_RBLK, b),
                         lambda j: (jnp.where(j == 0, 0, b_blk + j - 1), 0)),
        ],
        out_specs=[
            pl.BlockSpec((1, b), lambda j: (0, 0)),
            pl.BlockSpec((1, b), lambda j: (0, 0)),
            pl.BlockSpec((1, b), lambda j: (0, 0)),
        ],
        out_shape=[
            jax.ShapeDtypeStruct((1, b), jnp.float32),
            jax.ShapeDtypeStruct((1, b), jnp.float32),
            jax.ShapeDtypeStruct((1, b), jnp.float32),
        ],
    )(t_row, xT)


def _combine_body(n, t_ref, acc_tc, tacc_tc, zacc, acc_sc, tacc_sc, out_ref):
    s = _LS / (n - 2)
    k_const = _LS * math.log(s) + _CONF * math.log(_CONF)
    t = t_ref[...]
    r_total = acc_tc[...] + jnp.sum(acc_sc[...], axis=0, keepdims=True)
    xt = tacc_tc[...] + jnp.sum(tacc_sc[...], axis=0, keepdims=True)
    contrib = k_const - s * r_total + s * zacc[...] + (s - _CONF) * xt
    nonpad = t != 0
    total = jnp.sum(jnp.where(nonpad, contrib, jnp.zeros_like(contrib)))
    out_ref[...] = total.reshape(1, 1)


def kernel(output, target, extra_len):
    del extra_len
    b, n = output.shape
    xT = output.T
    tgt = target.astype(jnp.int32)
    t_row = tgt.reshape(1, b)
    acc_sc, tacc_sc = _sc_stream(xT, tgt)
    acc_tc, tacc_tc, zacc = _tc_stream(t_row, xT)
    body = functools.partial(_combine_body, n)
    res = pl.pallas_call(
        body,
        grid=(1,),
        in_specs=[
            pl.BlockSpec((1, b), lambda j: (0, 0)),
            pl.BlockSpec((1, b), lambda j: (0, 0)),
            pl.BlockSpec((1, b), lambda j: (0, 0)),
            pl.BlockSpec((1, b), lambda j: (0, 0)),
            pl.BlockSpec((_NW, b), lambda j: (0, 0)),
            pl.BlockSpec((_NW, b), lambda j: (0, 0)),
        ],
        out_specs=pl.BlockSpec((1, 1), lambda j: (0, 0)),
        out_shape=jax.ShapeDtypeStruct((1, 1), jnp.float32),
    )(t_row, acc_tc, tacc_tc, zacc, acc_sc, tacc_sc)
    return res[0, 0]


# speedup vs baseline: 3.9889x; 1.0110x over previous
import functools
import math

import jax
import jax.numpy as jnp
from jax import lax
from jax.experimental import pallas as pl
from jax.experimental.pallas import tpu as pltpu
from jax.experimental.pallas import tpu_sc as plsc

_LS = 0.1
_CONF = 1.0 - _LS
_RBLK = 4096
_A = 4096
_B = 47104
_CHR = 32

_NC = 2
_NS = 16
_L = 16
_NW = _NC * _NS


def _sc_body(b, n, xT, tgt, acc_out, tacc_out, tgt_v, buf, acc_v, tacc_v,
             sem0, sem1):
    w_per = (_B - _A) // _NW
    nchunks = w_per // _CHR
    ngroups = b // _L
    wid = lax.axis_index("s") * _NC + lax.axis_index("c")
    c_base = _A + wid * w_per

    pltpu.sync_copy(tgt, tgt_v)

    @pl.loop(0, ngroups)
    def _zero(j):
        off = j * _L
        acc_v[pl.ds(off, _L)] = jnp.zeros((_L,), jnp.float32)
        tacc_v[pl.ds(off, _L)] = jnp.zeros((_L,), jnp.float32)

    def start(k, bsel, sem):
        pltpu.make_async_copy(
            xT.at[pl.ds(c_base + k * _CHR, _CHR), :],
            buf.at[bsel], sem).start()

    def wait(bsel, sem):
        pltpu.make_async_copy(
            xT.at[pl.ds(0, _CHR), :], buf.at[bsel], sem).wait()

    def compute(bufref, c0):
        @pl.loop(0, ngroups)
        def _grp(j):
            off = j * _L
            a0 = bufref[0, pl.ds(off, _L)]
            a1 = bufref[1, pl.ds(off, _L)]
            a2 = bufref[2, pl.ds(off, _L)]
            a3 = bufref[3, pl.ds(off, _L)]
            for r in range(4, _CHR, 4):
                a0 = a0 + bufref[r, pl.ds(off, _L)]
                a1 = a1 + bufref[r + 1, pl.ds(off, _L)]
                a2 = a2 + bufref[r + 2, pl.ds(off, _L)]
                a3 = a3 + bufref[r + 3, pl.ds(off, _L)]
            acc_v[pl.ds(off, _L)] = (acc_v[pl.ds(off, _L)]
                                     + (a0 + a1) + (a2 + a3))

            t16 = tgt_v[pl.ds(off, _L)]
            rel = t16 - c0
            valid = (rel >= 0) & (rel < _CHR)
            relc = jnp.minimum(jnp.maximum(rel, 0), _CHR - 1)
            bcol = off + lax.iota(jnp.int32, _L)
            xt = plsc.load_gather(bufref, [relc, bcol])
            zero = jnp.zeros((_L,), jnp.float32)
            tacc_v[pl.ds(off, _L)] = (tacc_v[pl.ds(off, _L)]
                                      + jnp.where(valid, xt, zero))

    if nchunks > 0:
        start(0, 0, sem0)
    if nchunks > 1:
        start(1, 1, sem1)

    @pl.loop(0, nchunks, step=2)
    def pair(k):
        wait(0, sem0)
        compute(buf.at[0], c_base + k * _CHR)

        @pl.when(k + 2 < nchunks)
        def _():
            start(k + 2, 0, sem0)

        wait(1, sem1)
        compute(buf.at[1], c_base + (k + 1) * _CHR)

        @pl.when(k + 3 < nchunks)
        def _():
            start(k + 3, 1, sem1)

    pltpu.sync_copy(acc_v, acc_out.at[wid])
    pltpu.sync_copy(tacc_v, tacc_out.at[wid])


def _sc_stream(xT, tgt):
    n, b = xT.shape
    mesh = plsc.VectorSubcoreMesh(core_axis_name="c", subcore_axis_name="s",
                                  num_cores=_NC, num_subcores=_NS)
    body = functools.partial(_sc_body, b, n)
    return pl.kernel(
        body,
        out_type=(jax.ShapeDtypeStruct((_NW, b), jnp.float32),
                  jax.ShapeDtypeStruct((_NW, b), jnp.float32)),
        mesh=mesh,
        compiler_params=pltpu.CompilerParams(needs_layout_passes=False),
        scratch_types=[
            pltpu.VMEM((b,), jnp.int32),
            pltpu.VMEM((2, _CHR, b), jnp.float32),
            pltpu.VMEM((b,), jnp.float32),
            pltpu.VMEM((b,), jnp.float32),
            pltpu.SemaphoreType.DMA,
            pltpu.SemaphoreType.DMA,
        ],
    )(xT, tgt)


def _tc_body(n, t_ref, x_ref, acc_ref, tacc_ref, zacc_ref):
    j = pl.program_id(0)

    @pl.when(j == 0)
    def _init():
        acc_ref[...] = jnp.zeros_like(acc_ref)
        tacc_ref[...] = jnp.zeros_like(tacc_ref)

    x = x_ref[...]
    c0 = jnp.where(j == 0, 0, _B + (j - 1) * _RBLK)
    rid = c0 + jax.lax.broadcasted_iota(jnp.int32, x.shape, 0)
    t = t_ref[...]
    zero = jnp.zeros_like(x)
    xv = jnp.where(rid < n, x, zero)
    acc_ref[...] += jnp.sum(xv, axis=0, keepdims=True)
    tacc_ref[...] += jnp.sum(jnp.where(rid == t, x, zero), axis=0,
                             keepdims=True)

    @pl.when(j == 0)
    def _zrow():
        zacc_ref[...] = x[0:1, :]


def _tc_stream(t_row, xT):
    n, b = xT.shape
    nback = pl.cdiv(n - _B, _RBLK)
    nblocks = 1 + nback
    b_blk = _B // _RBLK
    body = functools.partial(_tc_body, n)
    return pl.pallas_call(
        body,
        grid=(nblocks,),
        in_specs=[
            pl.BlockSpec((1, b), lambda j: (0, 0)),
            pl.BlockSpec((_RBLK, b),
                         lambda j: (jnp.where(j == 0, 0, b_blk + j - 1), 0)),
        ],
        out_specs=[
            pl.BlockSpec((1, b), lambda j: (0, 0)),
            pl.BlockSpec((1, b), lambda j: (0, 0)),
            pl.BlockSpec((1, b), lambda j: (0, 0)),
        ],
        out_shape=[
            jax.ShapeDtypeStruct((1, b), jnp.float32),
            jax.ShapeDtypeStruct((1, b), jnp.float32),
            jax.ShapeDtypeStruct((1, b), jnp.float32),
        ],
    )(t_row, xT)


def _combine_body(n, t_ref, acc_tc, tacc_tc, zacc, acc_sc, tacc_sc, out_ref):
    s = _LS / (n - 2)
    k_const = _LS * math.log(s) + _CONF * math.log(_CONF)
    t = t_ref[...]
    r_total = acc_tc[...] + jnp.sum(acc_sc[...], axis=0, keepdims=True)
    xt = tacc_tc[...] + jnp.sum(tacc_sc[...], axis=0, keepdims=True)
    contrib = k_const - s * r_total + s * zacc[...] + (s - _CONF) * xt
    nonpad = t != 0
    total = jnp.sum(jnp.where(nonpad, contrib, jnp.zeros_like(contrib)))
    out_ref[...] = total.reshape(1, 1)


def kernel(output, target, extra_len):
    del extra_len
    b, n = output.shape
    xT = output.T
    tgt = target.astype(jnp.int32)
    t_row = tgt.reshape(1, b)
    acc_sc, tacc_sc = _sc_stream(xT, tgt)
    acc_tc, tacc_tc, zacc = _tc_stream(t_row, xT)
    body = functools.partial(_combine_body, n)
    res = pl.pallas_call(
        body,
        grid=(1,),
        in_specs=[
            pl.BlockSpec((1, b), lambda j: (0, 0)),
            pl.BlockSpec((1, b), lambda j: (0, 0)),
            pl.BlockSpec((1, b), lambda j: (0, 0)),
            pl.BlockSpec((1, b), lambda j: (0, 0)),
            pl.BlockSpec((_NW, b), lambda j: (0, 0)),
            pl.BlockSpec((_NW, b), lambda j: (0, 0)),
        ],
        out_specs=pl.BlockSpec((1, 1), lambda j: (0, 0)),
        out_shape=jax.ShapeDtypeStruct((1, 1), jnp.float32),
    )(t_row, acc_tc, tacc_tc, zacc, acc_sc, tacc_sc)
    return res[0, 0]
